# Initial kernel scaffold; baseline (speedup 1.0000x reference)
import jax
import jax.numpy as jnp
from jax.experimental import pallas as pl


def kernel(pred_probas, y_true):
    raise NotImplementedError("write your pallas kernel here")


# speedup vs baseline: 1.0048x; 1.0048x over previous
import functools

import numpy as np
import jax
import jax.numpy as jnp
from jax import lax
from jax.experimental import pallas as pl
from jax.experimental.pallas import tpu as pltpu
from jax.experimental.pallas import tpu_sc as plsc

_N_BINS = 15
_NC = 2
_NS = 16
_L = 16
_NW = _NC * _NS


@functools.cache
def _build_sc_hist(chunk):
    mesh = plsc.VectorSubcoreMesh(core_axis_name="c", subcore_axis_name="s")

    @functools.partial(
        pl.kernel,
        mesh=mesh,
        compiler_params=pltpu.CompilerParams(needs_layout_passes=False),
        out_type=jax.ShapeDtypeStruct((_NW, 3 * _L), jnp.float32),
        scratch_types=[
            pltpu.VMEM((chunk,), jnp.float32),
            pltpu.VMEM((chunk,), jnp.float32),
            pltpu.VMEM((_L,), jnp.float32),
            pltpu.VMEM((_L * _L,), jnp.float32),
            pltpu.VMEM((_L * _L,), jnp.float32),
            pltpu.VMEM((_L * _L,), jnp.float32),
            pltpu.VMEM((3 * _L,), jnp.float32),
        ],
    )
    def sc_hist(p_hbm, y_hbm, edges_hbm, out_hbm,
                p_v, y_v, e_v, cnt_t, sp_t, sy_t, res_v):
        wid = lax.axis_index("c") * _NS + lax.axis_index("s")
        base = wid * chunk
        pltpu.sync_copy(edges_hbm, e_v)
        pltpu.sync_copy(p_hbm.at[pl.ds(base, chunk)], p_v)
        pltpu.sync_copy(y_hbm.at[pl.ds(base, chunk)], y_v)

        zeros = jnp.zeros((_L,), jnp.float32)
        for k in range(_L):
            cnt_t[pl.ds(k * _L, _L)] = zeros
            sp_t[pl.ds(k * _L, _L)] = zeros
            sy_t[pl.ds(k * _L, _L)] = zeros

        lane_base = lax.iota(jnp.int32, _L) * _L
        ones = jnp.ones((_L,), jnp.float32)

        def body(i, carry):
            s = pl.ds(i * _L, _L)
            p = p_v[s]
            y = y_v[s]
            b0 = jnp.minimum((p * 15.0).astype(jnp.int32), _N_BINS - 1)
            lo = plsc.load_gather(e_v, [b0])
            hi = plsc.load_gather(e_v, [b0 + 1])
            b = jnp.where(p < lo, b0 - 1, b0)
            b = jnp.where(p >= hi, b + 1, b)
            idx = lane_base + b
            plsc.addupdate_scatter(cnt_t, [idx], ones)
            plsc.addupdate_scatter(sp_t, [idx], p)
            plsc.addupdate_scatter(sy_t, [idx], y)
            return carry

        lax.fori_loop(0, chunk // _L, body, jnp.int32(0))

        for j, tab in enumerate((cnt_t, sp_t, sy_t)):
            acc = tab[pl.ds(0, _L)]
            for k in range(1, _L):
                acc = acc + tab[pl.ds(k * _L, _L)]
            res_v[pl.ds(j * _L, _L)] = acc
        pltpu.sync_copy(res_v, out_hbm.at[wid])

    return sc_hist


@functools.cache
def _build_epilogue(n):
    def body(x_ref, o_ref):
        s = jnp.sum(x_ref[...], axis=0, keepdims=True)
        cnt = s[:, 0:_L]
        sp = s[:, _L:2 * _L]
        sy = s[:, 2 * _L:3 * _L]
        valid = lax.broadcasted_iota(jnp.int32, (1, _L), 1) < _N_BINS
        cnt = jnp.where(valid, cnt, 0.0)
        sp = jnp.where(valid, sp, 0.0)
        sy = jnp.where(valid, sy, 0.0)
        frac = jnp.sum(sy) / np.float32(n)
        e1 = 2.0 * frac
        e2 = 0.5 + frac
        safe = jnp.maximum(cnt, 1.0)
        p_mean = sp / safe
        y_mean = sy / safe
        lg = jnp.log(safe)
        nonempty = valid & (cnt > 0.0)
        w = jnp.where(nonempty, 0.5 * (jnp.exp(e1 * lg) + jnp.exp(e2 * lg)),
                      0.0)
        num = jnp.abs(p_mean - y_mean) * w
        o_ref[...] = (jnp.sum(num, axis=1, keepdims=True)
                      / jnp.sum(w, axis=1, keepdims=True))

    return pl.pallas_call(
        body, out_shape=jax.ShapeDtypeStruct((1, 1), jnp.float32))


def kernel(pred_probas, y_true):
    n = pred_probas.shape[0]
    unit = _NW * _L
    chunk = ((n + unit - 1) // unit) * _L
    n_pad = chunk * _NW
    p = jnp.pad(pred_probas, (0, n_pad - n), constant_values=2.0)
    y = jnp.pad(y_true, (0, n_pad - n), constant_values=0.0)
    edges = np.linspace(0.0, 1.0, _N_BINS + 1).astype(np.float32)
    edges[_N_BINS] = 1.01
    partials = _build_sc_hist(chunk)(p, y, jnp.asarray(edges))
    out = _build_epilogue(n)(partials)
    return out[0, 0]


# speedup vs baseline: 1.0834x; 1.0782x over previous
import functools

import numpy as np
import jax
import jax.numpy as jnp
from jax import lax
from jax.experimental import pallas as pl
from jax.experimental.pallas import tpu as pltpu
from jax.experimental.pallas import tpu_sc as plsc

_N_BINS = 15
_NC = 2
_NS = 16
_L = 16
_NW = _NC * _NS


@functools.cache
def _build_sc_hist(n):
    assert n % _L == 0
    big = ((n + _NW * _L - 1) // (_NW * _L)) * _L
    small = n - (_NW - 1) * big
    assert 0 < small <= big and small % _L == 0
    n_common = small // _L
    n_extra = (big - small) // _L
    mesh = plsc.VectorSubcoreMesh(core_axis_name="c", subcore_axis_name="s")

    @functools.partial(
        pl.kernel,
        mesh=mesh,
        compiler_params=pltpu.CompilerParams(needs_layout_passes=False),
        out_type=jax.ShapeDtypeStruct((_NW, 3 * _L), jnp.float32),
        scratch_types=[
            pltpu.VMEM((big,), jnp.float32),
            pltpu.VMEM((big,), jnp.float32),
            pltpu.VMEM((_L,), jnp.float32),
            pltpu.VMEM((_L * _L,), jnp.float32),
            pltpu.VMEM((_L * _L,), jnp.float32),
            pltpu.VMEM((_L * _L,), jnp.float32),
            pltpu.VMEM((3 * _L,), jnp.float32),
            pltpu.SemaphoreType.DMA,
            pltpu.SemaphoreType.DMA,
            pltpu.SemaphoreType.DMA,
        ],
    )
    def sc_hist(p_hbm, y_hbm, edges_hbm, out_hbm,
                p_v, y_v, e_v, cnt_t, sp_t, sy_t, res_v, sem_p, sem_y, sem_e):
        wid = lax.axis_index("c") * _NS + lax.axis_index("s")
        base = wid * big
        is_big = wid < _NW - 1

        cp_e = pltpu.async_copy(edges_hbm, e_v, sem_e)
        cp_p = pltpu.async_copy(
            p_hbm.at[pl.ds(base, small)], p_v.at[pl.ds(0, small)], sem_p)
        cp_y = pltpu.async_copy(
            y_hbm.at[pl.ds(base, small)], y_v.at[pl.ds(0, small)], sem_y)

        zeros = jnp.zeros((_L,), jnp.float32)
        for k in range(_L):
            cnt_t[pl.ds(k * _L, _L)] = zeros
            sp_t[pl.ds(k * _L, _L)] = zeros
            sy_t[pl.ds(k * _L, _L)] = zeros

        @pl.when(is_big)
        def _():
            pltpu.async_copy(
                p_hbm.at[pl.ds(base + small, big - small)],
                p_v.at[pl.ds(small, big - small)], sem_p)
            pltpu.async_copy(
                y_hbm.at[pl.ds(base + small, big - small)],
                y_v.at[pl.ds(small, big - small)], sem_y)

        lane_base = lax.iota(jnp.int32, _L) * _L
        ones = jnp.ones((_L,), jnp.float32)

        def step(v):
            s = pl.ds(v * _L, _L)
            p = p_v[s]
            y = y_v[s]
            b0 = jnp.minimum((p * 15.0).astype(jnp.int32), _N_BINS - 1)
            lo = plsc.load_gather(e_v, [b0])
            hi = plsc.load_gather(e_v, [b0 + 1])
            b = jnp.where(p < lo, b0 - 1, b0)
            b = jnp.where(p >= hi, b + 1, b)
            idx = lane_base + b
            plsc.addupdate_scatter(cnt_t, [idx], ones)
            plsc.addupdate_scatter(sp_t, [idx], p)
            plsc.addupdate_scatter(sy_t, [idx], y)

        cp_e.wait()
        cp_p.wait()
        cp_y.wait()

        unroll = 6
        assert n_common % unroll == 0

        def body(i, carry):
            for u in range(unroll):
                step(i * unroll + u)
            return carry

        lax.fori_loop(0, n_common // unroll, body, jnp.int32(0))

        @pl.when(is_big)
        def _():
            pltpu.make_async_copy(
                p_hbm.at[pl.ds(base + small, big - small)],
                p_v.at[pl.ds(small, big - small)], sem_p).wait()
            pltpu.make_async_copy(
                y_hbm.at[pl.ds(base + small, big - small)],
                y_v.at[pl.ds(small, big - small)], sem_y).wait()

            def body_extra(i, carry):
                step(n_common + i)
                return carry

            lax.fori_loop(0, n_extra, body_extra, jnp.int32(0))

        for j, tab in enumerate((cnt_t, sp_t, sy_t)):
            acc = tab[pl.ds(0, _L)]
            for k in range(1, _L):
                acc = acc + tab[pl.ds(k * _L, _L)]
            res_v[pl.ds(j * _L, _L)] = acc
        pltpu.sync_copy(res_v, out_hbm.at[wid])

    return sc_hist


@functools.cache
def _build_epilogue(n):
    def body(x_ref, o_ref):
        s = jnp.sum(x_ref[...], axis=0, keepdims=True)
        cnt = s[:, 0:_L]
        sp = s[:, _L:2 * _L]
        sy = s[:, 2 * _L:3 * _L]
        valid = lax.broadcasted_iota(jnp.int32, (1, _L), 1) < _N_BINS
        cnt = jnp.where(valid, cnt, 0.0)
        sp = jnp.where(valid, sp, 0.0)
        sy = jnp.where(valid, sy, 0.0)
        frac = jnp.sum(sy) / np.float32(n)
        e1 = 2.0 * frac
        e2 = 0.5 + frac
        safe = jnp.maximum(cnt, 1.0)
        p_mean = sp / safe
        y_mean = sy / safe
        lg = jnp.log(safe)
        nonempty = valid & (cnt > 0.0)
        w = jnp.where(nonempty, 0.5 * (jnp.exp(e1 * lg) + jnp.exp(e2 * lg)),
                      0.0)
        num = jnp.abs(p_mean - y_mean) * w
        o_ref[...] = (jnp.sum(num, axis=1, keepdims=True)
                      / jnp.sum(w, axis=1, keepdims=True))

    return pl.pallas_call(
        body, out_shape=jax.ShapeDtypeStruct((1, 1), jnp.float32))


def kernel(pred_probas, y_true):
    n = pred_probas.shape[0]
    edges = np.linspace(0.0, 1.0, _N_BINS + 1).astype(np.float32)
    edges[_N_BINS] = 1.01
    partials = _build_sc_hist(n)(pred_probas, y_true, jnp.asarray(edges))
    out = _build_epilogue(n)(partials)
    return out[0, 0]


# speedup vs baseline: 1.8522x; 1.7097x over previous
import functools

import numpy as np
import jax
import jax.numpy as jnp
from jax import lax
from jax.experimental import pallas as pl
from jax.experimental.pallas import tpu as pltpu
from jax.experimental.pallas import tpu_sc as plsc

_N_BINS = 15
_NC = 2
_NS = 16
_L = 16
_NW = _NC * _NS


@functools.cache
def _build_sc_hist(n):
    assert n % _L == 0
    big = ((n + _NW * _L - 1) // (_NW * _L)) * _L
    small = n - (_NW - 1) * big
    assert 0 < small <= big and small % _L == 0
    n_common = small // _L
    n_extra = (big - small) // _L
    mesh = plsc.VectorSubcoreMesh(core_axis_name="c", subcore_axis_name="s")

    @functools.partial(
        pl.kernel,
        mesh=mesh,
        compiler_params=pltpu.CompilerParams(needs_layout_passes=False),
        out_type=jax.ShapeDtypeStruct((_NW, 3 * _L), jnp.float32),
        scratch_types=[
            pltpu.VMEM((big,), jnp.float32),
            pltpu.VMEM((big,), jnp.float32),
            pltpu.VMEM((_L,), jnp.float32),
            pltpu.VMEM((_L * _L,), jnp.float32),
            pltpu.VMEM((_L * _L,), jnp.float32),
            pltpu.VMEM((_L * _L,), jnp.float32),
            pltpu.VMEM((3 * _L,), jnp.float32),
            pltpu.SemaphoreType.DMA,
            pltpu.SemaphoreType.DMA,
            pltpu.SemaphoreType.DMA,
        ],
    )
    def sc_hist(p_hbm, y_hbm, edges_hbm, out_hbm,
                p_v, y_v, e_v, cnt_t, sp_t, sy_t, res_v, sem_p, sem_y, sem_e):
        wid = lax.axis_index("c") * _NS + lax.axis_index("s")
        base = wid * big
        is_big = wid < _NW - 1

        cp_e = pltpu.async_copy(edges_hbm, e_v, sem_e)
        cp_p = pltpu.async_copy(
            p_hbm.at[pl.ds(base, small)], p_v.at[pl.ds(0, small)], sem_p)
        cp_y = pltpu.async_copy(
            y_hbm.at[pl.ds(base, small)], y_v.at[pl.ds(0, small)], sem_y)

        zeros = jnp.zeros((_L,), jnp.float32)
        for k in range(_L):
            cnt_t[pl.ds(k * _L, _L)] = zeros
            sp_t[pl.ds(k * _L, _L)] = zeros
            sy_t[pl.ds(k * _L, _L)] = zeros

        @pl.when(is_big)
        def _():
            pltpu.async_copy(
                p_hbm.at[pl.ds(base + small, big - small)],
                p_v.at[pl.ds(small, big - small)], sem_p)
            pltpu.async_copy(
                y_hbm.at[pl.ds(base + small, big - small)],
                y_v.at[pl.ds(small, big - small)], sem_y)

        lane_base = lax.iota(jnp.int32, _L) * _L
        ones = jnp.ones((_L,), jnp.float32)

        def step(v):
            s = pl.ds(v * _L, _L)
            p = p_v[s]
            y = y_v[s]
            b0 = jnp.minimum((p * 15.0).astype(jnp.int32), _N_BINS - 1)
            lo = plsc.load_gather(e_v, [b0])
            hi = plsc.load_gather(e_v, [b0 + 1])
            b = jnp.where(p < lo, b0 - 1, b0)
            b = jnp.where(p >= hi, b + 1, b)
            idx = lane_base + b
            plsc.addupdate_scatter(cnt_t, [idx], ones)
            plsc.addupdate_scatter(sp_t, [idx], p)
            plsc.addupdate_scatter(sy_t, [idx], y)

        def steps_interleaved(base_v, width):
            ss = [pl.ds((base_v + u) * _L, _L) for u in range(width)]
            ps = [p_v[s] for s in ss]
            b0s = [jnp.minimum((p * 15.0).astype(jnp.int32), _N_BINS - 1)
                   for p in ps]
            los = [plsc.load_gather(e_v, [b0]) for b0 in b0s]
            his = [plsc.load_gather(e_v, [b0 + 1]) for b0 in b0s]
            ys = [y_v[s] for s in ss]
            idxs = []
            for u in range(width):
                b = jnp.where(ps[u] < los[u], b0s[u] - 1, b0s[u])
                b = jnp.where(ps[u] >= his[u], b + 1, b)
                idxs.append(lane_base + b)
            for u in range(width):
                plsc.addupdate_scatter(cnt_t, [idxs[u]], ones)
                plsc.addupdate_scatter(sp_t, [idxs[u]], ps[u])
                plsc.addupdate_scatter(sy_t, [idxs[u]], ys[u])

        cp_e.wait()
        cp_p.wait()
        cp_y.wait()

        unroll = 6
        assert n_common % unroll == 0

        def body(i, carry):
            steps_interleaved(i * unroll, unroll)
            return carry

        lax.fori_loop(0, n_common // unroll, body, jnp.int32(0))

        @pl.when(is_big)
        def _():
            pltpu.make_async_copy(
                p_hbm.at[pl.ds(base + small, big - small)],
                p_v.at[pl.ds(small, big - small)], sem_p).wait()
            pltpu.make_async_copy(
                y_hbm.at[pl.ds(base + small, big - small)],
                y_v.at[pl.ds(small, big - small)], sem_y).wait()

            def body_extra(i, carry):
                step(n_common + i)
                return carry

            lax.fori_loop(0, n_extra, body_extra, jnp.int32(0))

        for j, tab in enumerate((cnt_t, sp_t, sy_t)):
            acc = tab[pl.ds(0, _L)]
            for k in range(1, _L):
                acc = acc + tab[pl.ds(k * _L, _L)]
            res_v[pl.ds(j * _L, _L)] = acc
        pltpu.sync_copy(res_v, out_hbm.at[wid])

    return sc_hist


@functools.cache
def _build_epilogue(n):
    def body(x_ref, o_ref):
        s = jnp.sum(x_ref[...], axis=0, keepdims=True)
        cnt = s[:, 0:_L]
        sp = s[:, _L:2 * _L]
        sy = s[:, 2 * _L:3 * _L]
        valid = lax.broadcasted_iota(jnp.int32, (1, _L), 1) < _N_BINS
        cnt = jnp.where(valid, cnt, 0.0)
        sp = jnp.where(valid, sp, 0.0)
        sy = jnp.where(valid, sy, 0.0)
        frac = jnp.sum(sy) / np.float32(n)
        e1 = 2.0 * frac
        e2 = 0.5 + frac
        safe = jnp.maximum(cnt, 1.0)
        p_mean = sp / safe
        y_mean = sy / safe
        lg = jnp.log(safe)
        nonempty = valid & (cnt > 0.0)
        w = jnp.where(nonempty, 0.5 * (jnp.exp(e1 * lg) + jnp.exp(e2 * lg)),
                      0.0)
        num = jnp.abs(p_mean - y_mean) * w
        o_ref[...] = (jnp.sum(num, axis=1, keepdims=True)
                      / jnp.sum(w, axis=1, keepdims=True))

    return pl.pallas_call(
        body, out_shape=jax.ShapeDtypeStruct((1, 1), jnp.float32))


def kernel(pred_probas, y_true):
    n = pred_probas.shape[0]
    edges = np.linspace(0.0, 1.0, _N_BINS + 1).astype(np.float32)
    edges[_N_BINS] = 1.01
    partials = _build_sc_hist(n)(pred_probas, y_true, jnp.asarray(edges))
    out = _build_epilogue(n)(partials)
    return out[0, 0]
